# speedup vs baseline: 5.1881x; 5.1881x over previous
import functools

import jax
import jax.numpy as jnp
from jax import lax
from jax.experimental import pallas as pl
from jax.experimental.pallas import tpu as pltpu
from jax.experimental.pallas import tpu_sc as plsc

_CHUNK = 128


def _gather_kernel(idx_hbm, table_hbm, out_hbm, idx_v, rows_v, sem,
                   *, per_worker, num_chunks, num_cores):
  wid = lax.axis_index("s") * num_cores + lax.axis_index("c")
  base = wid * per_worker

  def body(g, carry):
    off = base + g * _CHUNK
    pltpu.sync_copy(idx_hbm.at[pl.ds(off, _CHUNK)], idx_v)
    pltpu.async_copy(table_hbm.at[idx_v], rows_v, sem).wait()
    pltpu.sync_copy(rows_v, out_hbm.at[pl.ds(off, _CHUNK)])
    return carry

  lax.fori_loop(0, num_chunks, body, 0)


def kernel(pos_idxs, embeddings):
  batch, hist = pos_idxs.shape
  vocab, d = embeddings.shape
  n = batch * hist

  info = plsc.get_sparse_core_info()
  num_workers = info.num_cores * info.num_subcores
  assert n % (num_workers * _CHUNK) == 0
  per_worker = n // num_workers
  num_chunks = per_worker // _CHUNK

  flat_idx = pos_idxs.reshape(n).astype(jnp.int32)

  mesh = plsc.VectorSubcoreMesh(core_axis_name="c", subcore_axis_name="s")
  run = pl.kernel(
      functools.partial(
          _gather_kernel,
          per_worker=per_worker,
          num_chunks=num_chunks,
          num_cores=info.num_cores,
      ),
      mesh=mesh,
      out_type=jax.ShapeDtypeStruct((n, d), jnp.float32),
      scratch_types=[
          pltpu.VMEM((_CHUNK,), jnp.int32),
          pltpu.VMEM((_CHUNK, d), jnp.float32),
          pltpu.SemaphoreType.DMA,
      ],
  )
  out = run(flat_idx, embeddings)
  return out.reshape(batch, hist, d)


# speedup vs baseline: 9.1775x; 1.7690x over previous
import functools

import jax
import jax.numpy as jnp
from jax import lax
from jax.experimental import pallas as pl
from jax.experimental.pallas import tpu as pltpu
from jax.experimental.pallas import tpu_sc as plsc

_CHUNK = 128
_NBUF = 4
_AHEAD = 2


def _gather_kernel(idx_hbm, table_hbm, out_hbm, idx_v, rows, gsems, ssems,
                   *, per_worker, num_chunks, num_cores):
  wid = lax.axis_index("s") * num_cores + lax.axis_index("c")
  base = wid * per_worker

  def fire_gather(g, b):
    pltpu.async_copy(table_hbm.at[idx_v.at[g]], rows.at[b], gsems.at[b])

  def wait_gather(b):
    pltpu.make_async_copy(table_hbm.at[idx_v.at[0]], rows.at[b],
                          gsems.at[b]).wait()

  def fire_scatter(g, b):
    pltpu.async_copy(rows.at[b], out_hbm.at[pl.ds(base + g * _CHUNK, _CHUNK)],
                     ssems.at[b])

  def wait_scatter(b):
    pltpu.make_async_copy(rows.at[b], out_hbm.at[pl.ds(base, _CHUNK)],
                          ssems.at[b]).wait()

  pltpu.sync_copy(idx_hbm.at[wid], idx_v)

  fire_gather(0, 0)
  fire_gather(1, 1)

  for b in range(_NBUF):
    wait_gather(b)
    fire_scatter(b, b)
    if b < _AHEAD:
      fire_gather(b + _AHEAD, (b + _AHEAD) % _NBUF)
    else:
      wait_scatter((b + _AHEAD) % _NBUF)
      fire_gather(b + _AHEAD, (b + _AHEAD) % _NBUF)

  last_j = num_chunks // _NBUF - 1

  def body(j, carry):
    g0 = j * _NBUF
    for b in range(_NBUF):
      g = g0 + b
      wait_gather(b)
      fire_scatter(g, b)
      bh = (b + _AHEAD) % _NBUF
      if b < _NBUF - _AHEAD:
        wait_scatter(bh)
        fire_gather(g + _AHEAD, bh)
      else:
        @pl.when(j < last_j)
        def _():
          wait_scatter(bh)
          fire_gather(g + _AHEAD, bh)
    return carry

  lax.fori_loop(1, num_chunks // _NBUF, body, 0)

  for b in range(_NBUF):
    wait_scatter(b)


def kernel(pos_idxs, embeddings):
  batch, hist = pos_idxs.shape
  vocab, d = embeddings.shape
  n = batch * hist

  info = plsc.get_sparse_core_info()
  num_workers = info.num_cores * info.num_subcores
  assert n % (num_workers * _CHUNK) == 0
  per_worker = n // num_workers
  num_chunks = per_worker // _CHUNK
  assert num_chunks % _NBUF == 0

  flat_idx = pos_idxs.reshape(num_workers, num_chunks, _CHUNK).astype(jnp.int32)

  mesh = plsc.VectorSubcoreMesh(core_axis_name="c", subcore_axis_name="s")
  run = pl.kernel(
      functools.partial(
          _gather_kernel,
          per_worker=per_worker,
          num_chunks=num_chunks,
          num_cores=info.num_cores,
      ),
      mesh=mesh,
      out_type=jax.ShapeDtypeStruct((n, d), jnp.float32),
      scratch_types=[
          pltpu.VMEM((num_chunks, _CHUNK), jnp.int32),
          pltpu.VMEM((_NBUF, _CHUNK, d), jnp.float32),
          pltpu.SemaphoreType.DMA((_NBUF,)),
          pltpu.SemaphoreType.DMA((_NBUF,)),
      ],
  )
  out = run(flat_idx, embeddings)
  return out.reshape(batch, hist, d)


# speedup vs baseline: 9.1925x; 1.0016x over previous
import functools

import jax
import jax.numpy as jnp
from jax import lax
from jax.experimental import pallas as pl
from jax.experimental.pallas import tpu as pltpu
from jax.experimental.pallas import tpu_sc as plsc

_CHUNK = 128
_NBUF = 5
_AHEAD = 3


def _gather_kernel(idx_hbm, table_hbm, out_hbm, idx_v, rows, gsems, ssems,
                   *, per_worker, num_chunks, num_cores):
  wid = lax.axis_index("s") * num_cores + lax.axis_index("c")
  base = wid * per_worker

  def fire_gather(g, b):
    pltpu.async_copy(table_hbm.at[idx_v.at[g]], rows.at[b], gsems.at[b])

  def wait_gather(b):
    pltpu.make_async_copy(table_hbm.at[idx_v.at[0]], rows.at[b],
                          gsems.at[b]).wait()

  def fire_scatter(g, b):
    pltpu.async_copy(rows.at[b], out_hbm.at[pl.ds(base + g * _CHUNK, _CHUNK)],
                     ssems.at[b])

  def wait_scatter(b):
    pltpu.make_async_copy(rows.at[b], out_hbm.at[pl.ds(base, _CHUNK)],
                          ssems.at[b]).wait()

  pltpu.sync_copy(idx_hbm.at[wid], idx_v)

  for g in range(_AHEAD):
    fire_gather(g, g)

  for b in range(_NBUF):
    wait_gather(b)
    fire_scatter(b, b)
    if b + _AHEAD >= _NBUF:
      wait_scatter((b + _AHEAD) % _NBUF)
    fire_gather(b + _AHEAD, (b + _AHEAD) % _NBUF)

  last_j = num_chunks // _NBUF - 1

  def body(j, carry):
    g0 = j * _NBUF
    for b in range(_NBUF):
      g = g0 + b
      wait_gather(b)
      fire_scatter(g, b)
      bh = (b + _AHEAD) % _NBUF
      if b < _NBUF - _AHEAD:
        wait_scatter(bh)
        fire_gather(g + _AHEAD, bh)
      else:
        @pl.when(j < last_j)
        def _():
          wait_scatter(bh)
          fire_gather(g + _AHEAD, bh)
    return carry

  lax.fori_loop(1, num_chunks // _NBUF, body, 0)

  for b in range(_NBUF):
    wait_scatter(b)


def kernel(pos_idxs, embeddings):
  batch, hist = pos_idxs.shape
  vocab, d = embeddings.shape
  n = batch * hist

  info = plsc.get_sparse_core_info()
  num_workers = info.num_cores * info.num_subcores
  assert n % (num_workers * _CHUNK) == 0
  per_worker = n // num_workers
  num_chunks = per_worker // _CHUNK
  assert num_chunks % _NBUF == 0

  flat_idx = pos_idxs.reshape(num_workers, num_chunks, _CHUNK).astype(jnp.int32)

  mesh = plsc.VectorSubcoreMesh(core_axis_name="c", subcore_axis_name="s")
  run = pl.kernel(
      functools.partial(
          _gather_kernel,
          per_worker=per_worker,
          num_chunks=num_chunks,
          num_cores=info.num_cores,
      ),
      mesh=mesh,
      out_type=jax.ShapeDtypeStruct((n, d), jnp.float32),
      scratch_types=[
          pltpu.VMEM((num_chunks, _CHUNK), jnp.int32),
          pltpu.VMEM((_NBUF, _CHUNK, d), jnp.float32),
          pltpu.SemaphoreType.DMA((_NBUF,)),
          pltpu.SemaphoreType.DMA((_NBUF,)),
      ],
  )
  out = run(flat_idx, embeddings)
  return out.reshape(batch, hist, d)


# speedup vs baseline: 9.2073x; 1.0016x over previous
import functools

import jax
import jax.numpy as jnp
from jax import lax
from jax.experimental import pallas as pl
from jax.experimental.pallas import tpu as pltpu
from jax.experimental.pallas import tpu_sc as plsc

_CHUNK = 128
_GPB = 2
_ROWS = _CHUNK * _GPB
_NBUF = 2


def _gather_kernel(idx_hbm, table_hbm, out_hbm, idx_v, rows, gsems, ssems,
                   *, per_worker, num_blocks, num_cores):
  wid = lax.axis_index("s") * num_cores + lax.axis_index("c")
  base = wid * per_worker
  last_j = num_blocks // _NBUF - 1

  def fire_gathers(g, b):
    for i in range(_GPB):
      pltpu.async_copy(table_hbm.at[idx_v.at[g * _GPB + i]],
                       rows.at[b, pl.ds(i * _CHUNK, _CHUNK)], gsems.at[b])

  def wait_gathers(b):
    pltpu.make_async_copy(table_hbm.at[pl.ds(0, _ROWS)], rows.at[b],
                          gsems.at[b]).wait()

  def fire_scatter(g, b):
    pltpu.async_copy(rows.at[b], out_hbm.at[pl.ds(base + g * _ROWS, _ROWS)],
                     ssems.at[b])

  def wait_scatter(b):
    pltpu.make_async_copy(rows.at[b], out_hbm.at[pl.ds(base, _ROWS)],
                          ssems.at[b]).wait()

  pltpu.sync_copy(idx_hbm.at[wid], idx_v)
  fire_gathers(0, 0)

  def body(j, carry):
    wait_gathers(0)
    fire_scatter(j * 2, 0)

    @pl.when(j > 0)
    def _():
      wait_scatter(1)

    fire_gathers(j * 2 + 1, 1)

    wait_gathers(1)
    fire_scatter(j * 2 + 1, 1)

    @pl.when(j < last_j)
    def _():
      wait_scatter(0)
      fire_gathers(j * 2 + 2, 0)

    return carry

  lax.fori_loop(0, num_blocks // _NBUF, body, 0)

  wait_scatter(0)
  wait_scatter(1)


def kernel(pos_idxs, embeddings):
  batch, hist = pos_idxs.shape
  vocab, d = embeddings.shape
  n = batch * hist

  info = plsc.get_sparse_core_info()
  num_workers = info.num_cores * info.num_subcores
  assert n % (num_workers * _ROWS) == 0
  per_worker = n // num_workers
  num_blocks = per_worker // _ROWS
  assert num_blocks % _NBUF == 0

  flat_idx = pos_idxs.reshape(num_workers, per_worker // _CHUNK,
                              _CHUNK).astype(jnp.int32)

  mesh = plsc.VectorSubcoreMesh(core_axis_name="c", subcore_axis_name="s")
  run = pl.kernel(
      functools.partial(
          _gather_kernel,
          per_worker=per_worker,
          num_blocks=num_blocks,
          num_cores=info.num_cores,
      ),
      mesh=mesh,
      out_type=jax.ShapeDtypeStruct((n, d), jnp.float32),
      scratch_types=[
          pltpu.VMEM((per_worker // _CHUNK, _CHUNK), jnp.int32),
          pltpu.VMEM((_NBUF, _ROWS, d), jnp.float32),
          pltpu.SemaphoreType.DMA((_NBUF,)),
          pltpu.SemaphoreType.DMA((_NBUF,)),
      ],
  )
  out = run(flat_idx, embeddings)
  return out.reshape(batch, hist, d)
